# Initial kernel scaffold; baseline (speedup 1.0000x reference)
import jax
import jax.numpy as jnp
from jax.experimental import pallas as pl


def kernel(x, edge_index, W1, b1, Wm, bm, W2, b2):
    raise NotImplementedError("write your pallas kernel here")


# speedup vs baseline: 8.8782x; 8.8782x over previous
import functools

import jax
import jax.numpy as jnp
from jax import lax
from jax.experimental import pallas as pl
from jax.experimental.pallas import tpu as pltpu
from jax.experimental.pallas import tpu_sc as plsc

N = 10000
E = 320000
D = 128

NC = 2
NS = 16
NW = NC * NS
E_TILE = E // NW
CH = 80
NCH = E_TILE // CH
NP = 10240
N_TILE = NP // NS

_sc_mesh = plsc.VectorSubcoreMesh(core_axis_name="c", subcore_axis_name="s",
                                  num_cores=NC, num_subcores=NS)


def _zero_vmem(buf, width):
    zeros = jnp.zeros((16,), jnp.float32)

    def body(i, _):
        for j in range(width // 16):
            buf[i, j * 16:(j + 1) * 16] = zeros
        return _

    lax.fori_loop(0, buf.shape[0], body, None)


@functools.partial(
    pl.kernel,
    out_type=jax.ShapeDtypeStruct((NC, NS, N_TILE, D), jnp.float32),
    mesh=_sc_mesh,
    scratch_types=[
        pltpu.VMEM((NCH, CH), jnp.int32),
        pltpu.VMEM((NCH, CH), jnp.int32),
        pltpu.VMEM((CH, D), jnp.float32),
        pltpu.SemaphoreType.DMA,
        pltpu.VMEM_SHARED((NP, D), jnp.float32),
    ],
)
def _sc_scatter_rows(y_hbm, src_hbm, dst_hbm, out_hbm,
                     idx_s, idx_d, rows, sem, acc):
    cid = lax.axis_index("c")
    sid = lax.axis_index("s")
    tile = cid * NS + sid

    pltpu.sync_copy(src_hbm.at[tile], idx_s)
    pltpu.sync_copy(dst_hbm.at[tile], idx_d)

    _zero_vmem(rows, D)
    for r in range(N_TILE // CH):
        pltpu.sync_copy(rows, acc.at[pl.ds(sid * N_TILE + r * CH, CH)])
    plsc.subcore_barrier()

    def body(c, _):
        pltpu.async_copy(y_hbm.at[idx_s.at[c]], rows, sem).wait()
        pltpu.sync_copy(rows, acc.at[idx_d.at[c]], add=True)
        return _

    lax.fori_loop(0, NCH, body, None)
    plsc.subcore_barrier()

    pltpu.sync_copy(acc.at[pl.ds(sid * N_TILE, N_TILE)],
                    out_hbm.at[cid, sid])


_RB = 1000
_NRB = N // _RB


def _prep_body(deg_ref, x_ref, dinv_ref, y_ref):
    d = deg_ref[0, :, 0:1] + deg_ref[1, :, 0:1]
    dinv = jnp.where(d > 0, lax.rsqrt(jnp.where(d > 0, d, 1.0)), 0.0)
    dinv_ref[...] = dinv
    y_ref[...] = x_ref[...] * dinv


_prep = pl.pallas_call(
    _prep_body,
    grid=(_NRB,),
    in_specs=[
        pl.BlockSpec((NC, _RB, D), lambda i: (0, i, 0)),
        pl.BlockSpec((_RB, D), lambda i: (i, 0)),
    ],
    out_specs=[
        pl.BlockSpec((_RB, 1), lambda i: (i, 0)),
        pl.BlockSpec((_RB, D), lambda i: (i, 0)),
    ],
    out_shape=[
        jax.ShapeDtypeStruct((N, 1), jnp.float32),
        jax.ShapeDtypeStruct((N, D), jnp.float32),
    ],
)


def _mid_body(u_ref, dinv_ref, tx1_ref, y1_ref):
    dinv = dinv_ref[...]
    tx1 = -(dinv * (u_ref[0] + u_ref[1]))
    tx1_ref[...] = tx1
    y1_ref[...] = dinv * tx1


_mid = pl.pallas_call(
    _mid_body,
    grid=(_NRB,),
    in_specs=[
        pl.BlockSpec((NC, _RB, D), lambda i: (0, i, 0)),
        pl.BlockSpec((_RB, 1), lambda i: (i, 0)),
    ],
    out_specs=[
        pl.BlockSpec((_RB, D), lambda i: (i, 0)),
        pl.BlockSpec((_RB, D), lambda i: (i, 0)),
    ],
    out_shape=[
        jax.ShapeDtypeStruct((N, D), jnp.float32),
        jax.ShapeDtypeStruct((N, D), jnp.float32),
    ],
)


def _fin_body(u_ref, tx0_ref, tx1_ref, dinv_ref, w_ref, b_ref, *out_refs,
              relu):
    dinv = dinv_ref[...]
    tx0 = tx0_ref[...]
    tx2 = -2.0 * dinv * (u_ref[0] + u_ref[1]) - tx0
    out = (jnp.dot(tx0, w_ref[0], preferred_element_type=jnp.float32)
           + jnp.dot(tx1_ref[...], w_ref[1], preferred_element_type=jnp.float32)
           + jnp.dot(tx2, w_ref[2], preferred_element_type=jnp.float32)
           + b_ref[...])
    if relu:
        out = jnp.maximum(out, 0.0)
        out_refs[0][...] = out
        out_refs[1][...] = dinv * out
    else:
        out_refs[0][...] = out


def _make_fin(relu):
    out_specs = [pl.BlockSpec((_RB, D), lambda i: (i, 0))]
    out_shape = [jax.ShapeDtypeStruct((N, D), jnp.float32)]
    if relu:
        out_specs = out_specs + [pl.BlockSpec((_RB, D), lambda i: (i, 0))]
        out_shape = out_shape + [jax.ShapeDtypeStruct((N, D), jnp.float32)]
    return pl.pallas_call(
        functools.partial(_fin_body, relu=relu),
        grid=(_NRB,),
        in_specs=[
            pl.BlockSpec((NC, _RB, D), lambda i: (0, i, 0)),
            pl.BlockSpec((_RB, D), lambda i: (i, 0)),
            pl.BlockSpec((_RB, D), lambda i: (i, 0)),
            pl.BlockSpec((_RB, 1), lambda i: (i, 0)),
            pl.BlockSpec((3, D, D), lambda i: (0, 0, 0)),
            pl.BlockSpec((1, D), lambda i: (0, 0)),
        ],
        out_specs=out_specs,
        out_shape=out_shape,
    )


_fin_relu = _make_fin(True)
_fin_last = _make_fin(False)


def kernel(x, edge_index, W1, b1, Wm, bm, W2, b2):
    ei = edge_index.astype(jnp.int32)
    src_t = ei[0].reshape(NW, NCH, CH)
    dst_t = ei[1].reshape(NW, NCH, CH)

    ones_nd = jnp.ones((N, D), jnp.float32)
    deg = _sc_scatter_rows(ones_nd, src_t, src_t).reshape(NC, NP, D)
    dinv, y0 = _prep(deg, x)

    def layer(tx0, y_in, W, b, fin):
        u1 = _sc_scatter_rows(y_in, src_t, dst_t).reshape(NC, NP, D)
        tx1, y1 = _mid(u1, dinv)
        u2 = _sc_scatter_rows(y1, src_t, dst_t).reshape(NC, NP, D)
        return fin(u2, tx0, tx1, dinv, W, b.reshape(1, D))

    h1, ya = layer(x, y0, W1, b1, _fin_relu)
    h2, yb = layer(h1, ya, Wm, bm, _fin_relu)
    (out,) = layer(h2, yb, W2, b2, _fin_last)
    return out


# speedup vs baseline: 9.7349x; 1.0965x over previous
import functools

import jax
import jax.numpy as jnp
from jax import lax
from jax.experimental import pallas as pl
from jax.experimental.pallas import tpu as pltpu
from jax.experimental.pallas import tpu_sc as plsc

N = 10000
E = 320000
D = 128

NC = 2
NS = 16
NW = NC * NS
E_TILE = E // NW
CH = 80
NCH = E_TILE // CH
NP = 10240
N_TILE = NP // NS

_sc_mesh = plsc.VectorSubcoreMesh(core_axis_name="c", subcore_axis_name="s",
                                  num_cores=NC, num_subcores=NS)


def _zero_vmem(buf, width):
    zeros = jnp.zeros((16,), jnp.float32)

    def body(i, _):
        for j in range(width // 16):
            buf[i, j * 16:(j + 1) * 16] = zeros
        return _

    lax.fori_loop(0, buf.shape[0], body, None)


@functools.partial(
    pl.kernel,
    out_type=jax.ShapeDtypeStruct((NC, NS, N_TILE, D), jnp.float32),
    mesh=_sc_mesh,
    scratch_types=[
        pltpu.VMEM((NCH, CH), jnp.int32),
        pltpu.VMEM((NCH, CH), jnp.int32),
        pltpu.VMEM((CH, D), jnp.float32),
        pltpu.SemaphoreType.DMA,
        pltpu.VMEM_SHARED((NP, D), jnp.float32),
    ],
)
def _sc_scatter_rows(y_hbm, src_hbm, dst_hbm, out_hbm,
                     idx_s, idx_d, rows, sem, acc):
    cid = lax.axis_index("c")
    sid = lax.axis_index("s")
    tile = cid * NS + sid

    pltpu.sync_copy(src_hbm.at[tile], idx_s)
    pltpu.sync_copy(dst_hbm.at[tile], idx_d)

    _zero_vmem(rows, D)
    for r in range(N_TILE // CH):
        pltpu.sync_copy(rows, acc.at[pl.ds(sid * N_TILE + r * CH, CH)])
    plsc.subcore_barrier()

    def body(c, _):
        pltpu.async_copy(y_hbm.at[idx_s.at[c]], rows, sem).wait()
        pltpu.sync_copy(rows, acc.at[idx_d.at[c]], add=True)
        return _

    lax.fori_loop(0, NCH, body, None)
    plsc.subcore_barrier()

    pltpu.sync_copy(acc.at[pl.ds(sid * N_TILE, N_TILE)],
                    out_hbm.at[cid, sid])


DW = 128


@functools.partial(
    pl.kernel,
    out_type=jax.ShapeDtypeStruct((NC, NS, N_TILE, DW), jnp.float32),
    mesh=_sc_mesh,
    scratch_types=[
        pltpu.VMEM((NCH, CH), jnp.int32),
        pltpu.VMEM((CH, DW), jnp.float32),
        pltpu.VMEM_SHARED((NP, DW), jnp.float32),
    ],
)
def _sc_degree(src_hbm, out_hbm, idx_s, rows, acc):
    cid = lax.axis_index("c")
    sid = lax.axis_index("s")
    tile = cid * NS + sid
    pltpu.sync_copy(src_hbm.at[tile], idx_s)

    _zero_vmem(rows, DW)
    for r in range(N_TILE // CH):
        pltpu.sync_copy(rows, acc.at[pl.ds(sid * N_TILE + r * CH, CH)])
    plsc.subcore_barrier()

    ones = jnp.ones((16,), jnp.float32)

    def fill(i, _):
        for j in range(DW // 16):
            rows[i, j * 16:(j + 1) * 16] = ones
        return _

    lax.fori_loop(0, CH, fill, None)

    def body(c, _):
        pltpu.sync_copy(rows, acc.at[idx_s.at[c]], add=True)
        return _

    lax.fori_loop(0, NCH, body, None)
    plsc.subcore_barrier()
    pltpu.sync_copy(acc.at[pl.ds(sid * N_TILE, N_TILE)], out_hbm.at[cid, sid])


_RB = 1000
_NRB = N // _RB


def _prep_body(deg_ref, x_ref, dinv_ref, y_ref):
    d = deg_ref[0, :, 0:1] + deg_ref[1, :, 0:1]
    dinv = jnp.where(d > 0, lax.rsqrt(jnp.where(d > 0, d, 1.0)), 0.0)
    dinv_ref[...] = dinv
    y_ref[...] = x_ref[...] * dinv


_prep = pl.pallas_call(
    _prep_body,
    grid=(_NRB,),
    in_specs=[
        pl.BlockSpec((NC, _RB, DW), lambda i: (0, i, 0)),
        pl.BlockSpec((_RB, D), lambda i: (i, 0)),
    ],
    out_specs=[
        pl.BlockSpec((_RB, 1), lambda i: (i, 0)),
        pl.BlockSpec((_RB, D), lambda i: (i, 0)),
    ],
    out_shape=[
        jax.ShapeDtypeStruct((N, 1), jnp.float32),
        jax.ShapeDtypeStruct((N, D), jnp.float32),
    ],
)


def _mid_body(u_ref, dinv_ref, tx1_ref, y1_ref):
    dinv = dinv_ref[...]
    tx1 = -(dinv * (u_ref[0] + u_ref[1]))
    tx1_ref[...] = tx1
    y1_ref[...] = dinv * tx1


_mid = pl.pallas_call(
    _mid_body,
    grid=(_NRB,),
    in_specs=[
        pl.BlockSpec((NC, _RB, D), lambda i: (0, i, 0)),
        pl.BlockSpec((_RB, 1), lambda i: (i, 0)),
    ],
    out_specs=[
        pl.BlockSpec((_RB, D), lambda i: (i, 0)),
        pl.BlockSpec((_RB, D), lambda i: (i, 0)),
    ],
    out_shape=[
        jax.ShapeDtypeStruct((N, D), jnp.float32),
        jax.ShapeDtypeStruct((N, D), jnp.float32),
    ],
)


def _fin_body(u_ref, tx0_ref, tx1_ref, dinv_ref, w_ref, b_ref, *out_refs,
              relu):
    dinv = dinv_ref[...]
    tx0 = tx0_ref[...]
    tx2 = -2.0 * dinv * (u_ref[0] + u_ref[1]) - tx0
    out = (jnp.dot(tx0, w_ref[0], preferred_element_type=jnp.float32)
           + jnp.dot(tx1_ref[...], w_ref[1], preferred_element_type=jnp.float32)
           + jnp.dot(tx2, w_ref[2], preferred_element_type=jnp.float32)
           + b_ref[...])
    if relu:
        out = jnp.maximum(out, 0.0)
        out_refs[0][...] = out
        out_refs[1][...] = dinv * out
    else:
        out_refs[0][...] = out


def _make_fin(relu):
    out_specs = [pl.BlockSpec((_RB, D), lambda i: (i, 0))]
    out_shape = [jax.ShapeDtypeStruct((N, D), jnp.float32)]
    if relu:
        out_specs = out_specs + [pl.BlockSpec((_RB, D), lambda i: (i, 0))]
        out_shape = out_shape + [jax.ShapeDtypeStruct((N, D), jnp.float32)]
    return pl.pallas_call(
        functools.partial(_fin_body, relu=relu),
        grid=(_NRB,),
        in_specs=[
            pl.BlockSpec((NC, _RB, D), lambda i: (0, i, 0)),
            pl.BlockSpec((_RB, D), lambda i: (i, 0)),
            pl.BlockSpec((_RB, D), lambda i: (i, 0)),
            pl.BlockSpec((_RB, 1), lambda i: (i, 0)),
            pl.BlockSpec((3, D, D), lambda i: (0, 0, 0)),
            pl.BlockSpec((1, D), lambda i: (0, 0)),
        ],
        out_specs=out_specs,
        out_shape=out_shape,
    )


_fin_relu = _make_fin(True)
_fin_last = _make_fin(False)


def kernel(x, edge_index, W1, b1, Wm, bm, W2, b2):
    ei = edge_index.astype(jnp.int32)
    src_t = ei[0].reshape(NW, NCH, CH)
    dst_t = ei[1].reshape(NW, NCH, CH)

    deg = _sc_degree(src_t).reshape(NC, NP, DW)
    dinv, y0 = _prep(deg, x)

    def layer(tx0, y_in, W, b, fin):
        u1 = _sc_scatter_rows(y_in, src_t, dst_t).reshape(NC, NP, D)
        tx1, y1 = _mid(u1, dinv)
        u2 = _sc_scatter_rows(y1, src_t, dst_t).reshape(NC, NP, D)
        return fin(u2, tx0, tx1, dinv, W, b.reshape(1, D))

    h1, ya = layer(x, y0, W1, b1, _fin_relu)
    h2, yb = layer(h1, ya, Wm, bm, _fin_relu)
    (out,) = layer(h2, yb, W2, b2, _fin_last)
    return out
